# baseline (speedup 1.0000x reference)
import functools

import jax
import jax.numpy as jnp
from jax import lax
from jax.experimental import pallas as pl
from jax.experimental.pallas import tpu as pltpu
from jax.experimental.pallas import tpu_sc as plsc

NODES = 100000
DIM = 128
B = 16384

_NC = 2
_NS = 16
_NW = _NC * _NS
_BPW = B // _NW
_CH = 256
_NCH = _BPW // _CH

_mesh = plsc.VectorSubcoreMesh(core_axis_name="c", subcore_axis_name="s")


@functools.partial(
    pl.kernel,
    mesh=_mesh,
    out_type=jax.ShapeDtypeStruct((B, DIM), jnp.float32),
    scratch_types=[
        pltpu.VMEM((_BPW,), jnp.int32),
        pltpu.VMEM((_NCH, _CH, DIM), jnp.float32),
        pltpu.SemaphoreType.DMA,
        pltpu.SemaphoreType.DMA,
        pltpu.SemaphoreType.DMA,
    ],
)
def _gather(table_hbm, idx_hbm, out_hbm, idx_v, rows_v, g0, g1, osem):
    wid = lax.axis_index("s") * _NC + lax.axis_index("c")
    base = wid * _BPW
    gsem = (g0, g1)
    pltpu.sync_copy(idx_hbm.at[pl.ds(base, _BPW)], idx_v)
    gat_h = [
        pltpu.async_copy(
            table_hbm.at[idx_v.at[pl.ds(c * _CH, _CH)]], rows_v.at[c], gsem[c]
        )
        for c in range(_NCH)
    ]
    out_h = []
    for c in range(_NCH):
        gat_h[c].wait()
        out_h.append(
            pltpu.async_copy(
                rows_v.at[c], out_hbm.at[pl.ds(base + c * _CH, _CH)], osem
            )
        )
    for h in out_h:
        h.wait()


def kernel(x, embedding):
    return _gather(embedding, x.astype(jnp.int32))
